# Initial kernel scaffold; baseline (speedup 1.0000x reference)
import jax
import jax.numpy as jnp
from jax.experimental import pallas as pl


def kernel(positions, numbers):
    raise NotImplementedError("write your pallas kernel here")


# speedup vs baseline: 7.8739x; 7.8739x over previous
import functools

import jax
import jax.numpy as jnp
from jax.experimental import pallas as pl
from jax.experimental.pallas import tpu as pltpu

K = 9
N = 7000
R = 128
NPAD = 7168
BIG = 2**30
INF = float("inf")


def _knn_block(p_ref, pt_ref, idx_ref, val_ref):
    pb = p_ref[...]
    pt = pt_ref[...]
    g = jax.lax.dot_general(pb, pt, (((1,), (0,)), ((), ())),
                            preferred_element_type=jnp.float32)
    sqi = jnp.sum(pb * pb, axis=1, keepdims=True)
    sqj = jnp.sum(pt * pt, axis=0, keepdims=True)
    col = jax.lax.broadcasted_iota(jnp.int32, (R, NPAD), 1)
    d = sqi + sqj - 2.0 * g
    d = jnp.where(col < N, d, INF)

    idxs = []
    vals = []
    for _ in range(K):
        vmin = jnp.min(d, axis=1, keepdims=True)
        cand = jnp.where(d == vmin, col, BIG)
        imin = jnp.min(cand, axis=1, keepdims=True)
        d = jnp.where(col == imin, INF, d)
        idxs.append(imin)
        vals.append(vmin)
    idx_ref[...] = jnp.concatenate(idxs, axis=1)
    val_ref[...] = jnp.concatenate(vals, axis=1)


@functools.partial(jax.jit, static_argnames=())
def _knn(p_pad, pt_pad):
    nblocks = p_pad.shape[0] // R
    idx, val = pl.pallas_call(
        _knn_block,
        grid=(nblocks,),
        in_specs=[
            pl.BlockSpec((R, 8), lambda i: (i, 0)),
            pl.BlockSpec((8, NPAD), lambda i: (0, 0)),
        ],
        out_specs=[
            pl.BlockSpec((R, K), lambda i: (i, 0)),
            pl.BlockSpec((R, K), lambda i: (i, 0)),
        ],
        out_shape=[
            jax.ShapeDtypeStruct((p_pad.shape[0], K), jnp.int32),
            jax.ShapeDtypeStruct((p_pad.shape[0], K), jnp.float32),
        ],
        compiler_params=pltpu.CompilerParams(
            dimension_semantics=("parallel",),
        ),
    )(p_pad, pt_pad)
    return idx, val


def kernel(positions, numbers):
    n = positions.shape[0]
    nrows = ((n + R - 1) // R) * R
    p_pad = jnp.zeros((nrows, 8), jnp.float32).at[:n, :3].set(positions)
    pt_pad = jnp.zeros((8, NPAD), jnp.float32).at[:3, :n].set(positions.T)
    idx, val = _knn(p_pad, pt_pad)
    src = idx[:n].reshape(-1)
    dst = jnp.repeat(jnp.arange(n, dtype=src.dtype), K)
    return src, dst, numbers, val[:n]
